# Initial kernel scaffold; baseline (speedup 1.0000x reference)
import jax
import jax.numpy as jnp
from jax.experimental import pallas as pl


def kernel(pos, x_0, x_1, cell_1, adj_0_0, adj_0_1, adj_1_1, params):
    raise NotImplementedError("write your pallas kernel here")


# speedup vs baseline: 4.7396x; 4.7396x over previous
import functools

import jax
import jax.numpy as jnp
from jax import lax
from jax.experimental import pallas as pl
from jax.experimental.pallas import tpu as pltpu
from jax.experimental.pallas import tpu_sc as plsc

N0 = 10000
N1 = 20000
H = 128
E00 = 320000
E01 = 160000
E11 = 160000
E00P = 327680
E01P = 163840
E11P = 163840
NC = 2
NS = 16
NW = NC * NS
N1P = 20480
F32 = jnp.float32


def _vsmesh():
    return plsc.VectorSubcoreMesh(
        core_axis_name="c", subcore_axis_name="s", num_cores=NC, num_subcores=NS)


def _geom_call(px, py, pz, c0, c1, c2, c3):
    cpw = N1P // NW

    @functools.partial(
        pl.kernel,
        out_type=[jax.ShapeDtypeStruct((N1P,), F32)] * 4,
        mesh=_vsmesh(),
        compiler_params=pltpu.CompilerParams(needs_layout_passes=False, use_tc_tiling_on_sc=False),
        scratch_types=(
            [pltpu.VMEM((N0,), F32)] * 3
            + [pltpu.VMEM((cpw,), jnp.int32)] * 4
            + [pltpu.VMEM((cpw,), F32)] * 4
        ),
    )
    def k(px_h, py_h, pz_h, c0_h, c1_h, c2_h, c3_h,
          ocx_h, ocy_h, ocz_h, om_h,
          pxv, pyv, pzv, i0, i1, i2, i3, ox, oy, oz, om):
        wid = lax.axis_index("s") * NC + lax.axis_index("c")
        base = wid * cpw
        pltpu.sync_copy(px_h, pxv)
        pltpu.sync_copy(py_h, pyv)
        pltpu.sync_copy(pz_h, pzv)
        pltpu.sync_copy(c0_h.at[pl.ds(base, cpw)], i0)
        pltpu.sync_copy(c1_h.at[pl.ds(base, cpw)], i1)
        pltpu.sync_copy(c2_h.at[pl.ds(base, cpw)], i2)
        pltpu.sync_copy(c3_h.at[pl.ds(base, cpw)], i3)

        def body(t, carry):
            o = pl.multiple_of(t * 16, 16)
            idx = [i0[pl.ds(o, 16)], i1[pl.ds(o, 16)],
                   i2[pl.ds(o, 16)], i3[pl.ds(o, 16)]]
            xs = [plsc.load_gather(pxv, [a]) for a in idx]
            ys = [plsc.load_gather(pyv, [a]) for a in idx]
            zs = [plsc.load_gather(pzv, [a]) for a in idx]
            ox[pl.ds(o, 16)] = (xs[0] + xs[1] + xs[2] + xs[3]) * 0.25
            oy[pl.ds(o, 16)] = (ys[0] + ys[1] + ys[2] + ys[3]) * 0.25
            oz[pl.ds(o, 16)] = (zs[0] + zs[1] + zs[2] + zs[3]) * 0.25
            m = jnp.zeros((16,), F32)
            for p in range(4):
                for q in range(p + 1, 4):
                    dx = xs[p] - xs[q]
                    dy = ys[p] - ys[q]
                    dz = zs[p] - zs[q]
                    m = jnp.maximum(m, dx * dx + dy * dy + dz * dz)
            om[pl.ds(o, 16)] = m
            return carry

        lax.fori_loop(0, cpw // 16, body, 0)
        pltpu.sync_copy(ox, ocx_h.at[pl.ds(base, cpw)])
        pltpu.sync_copy(oy, ocy_h.at[pl.ds(base, cpw)])
        pltpu.sync_copy(oz, ocz_h.at[pl.ds(base, cpw)])
        pltpu.sync_copy(om, om_h.at[pl.ds(base, cpw)])

    return k(px, py, pz, c0, c1, c2, c3)


def _edge_feat_call(ep, src_dim, dst_dim, tabs0, tabs1, m1, s_idx, r_idx):
    epw = ep // NW
    C = 1280
    nchunks = epw // C
    n_out = 1 + (src_dim == 1) + (dst_dim == 1)

    src_tabs = tabs0 if src_dim == 0 else tabs1
    dst_tabs = tabs0 if dst_dim == 0 else tabs1
    n_src = N0 if src_dim == 0 else N1
    n_dst = N0 if dst_dim == 0 else N1
    same = src_dim == dst_dim

    scratch = [pltpu.VMEM((n_src,), F32)] * 3
    if not same:
        scratch += [pltpu.VMEM((n_dst,), F32)] * 3
    need_m1 = (src_dim == 1) or (dst_dim == 1)
    if need_m1:
        scratch += [pltpu.VMEM((N1,), F32)]
    scratch += [pltpu.VMEM((C,), jnp.int32)] * 2
    scratch += [pltpu.VMEM((C,), F32)] * n_out

    ins = list(src_tabs) + ([] if same else list(dst_tabs))
    if need_m1:
        ins.append(m1)
    ins += [s_idx, r_idx]

    @functools.partial(
        pl.kernel,
        out_type=[jax.ShapeDtypeStruct((ep,), F32)] * n_out,
        mesh=_vsmesh(),
        compiler_params=pltpu.CompilerParams(needs_layout_passes=False, use_tc_tiling_on_sc=False),
        scratch_types=scratch,
    )
    def k(*refs):
        pos = 0
        sx_h, sy_h, sz_h = refs[pos:pos + 3]; pos += 3
        if not same:
            dx_h, dy_h, dz_h = refs[pos:pos + 3]; pos += 3
        else:
            dx_h, dy_h, dz_h = sx_h, sy_h, sz_h
        if need_m1:
            m1_h = refs[pos]; pos += 1
        s_h, r_h = refs[pos:pos + 2]; pos += 2
        out_hs = refs[pos:pos + n_out]; pos += n_out
        sxv, syv, szv = refs[pos:pos + 3]; pos += 3
        if not same:
            dxv, dyv, dzv = refs[pos:pos + 3]; pos += 3
        else:
            dxv, dyv, dzv = sxv, syv, szv
        if need_m1:
            m1v = refs[pos]; pos += 1
        sv, rv = refs[pos:pos + 2]; pos += 2
        obufs = refs[pos:pos + n_out]

        wid = lax.axis_index("s") * NC + lax.axis_index("c")
        wbase = wid * epw
        pltpu.sync_copy(sx_h, sxv)
        pltpu.sync_copy(sy_h, syv)
        pltpu.sync_copy(sz_h, szv)
        if not same:
            pltpu.sync_copy(dx_h, dxv)
            pltpu.sync_copy(dy_h, dyv)
            pltpu.sync_copy(dz_h, dzv)
        if need_m1:
            pltpu.sync_copy(m1_h, m1v)

        for cc in range(nchunks):
            base = wbase + cc * C
            pltpu.sync_copy(s_h.at[pl.ds(base, C)], sv)
            pltpu.sync_copy(r_h.at[pl.ds(base, C)], rv)

            def body(t, carry):
                o = pl.multiple_of(t * 16, 16)
                si = sv[pl.ds(o, 16)]
                ri = rv[pl.ds(o, 16)]
                ax = plsc.load_gather(sxv, [si])
                ay = plsc.load_gather(syv, [si])
                az = plsc.load_gather(szv, [si])
                bx = plsc.load_gather(dxv, [ri])
                by = plsc.load_gather(dyv, [ri])
                bz = plsc.load_gather(dzv, [ri])
                ex = ax - bx
                ey = ay - by
                ez = az - bz
                ob = 0
                obufs[ob][pl.ds(o, 16)] = ex * ex + ey * ey + ez * ez
                ob += 1
                if src_dim == 1:
                    obufs[ob][pl.ds(o, 16)] = plsc.load_gather(m1v, [si])
                    ob += 1
                if dst_dim == 1:
                    obufs[ob][pl.ds(o, 16)] = plsc.load_gather(m1v, [ri])
                return carry

            lax.fori_loop(0, C // 16, body, 0)
            for b, oh in zip(obufs, out_hs):
                pltpu.sync_copy(b, oh.at[pl.ds(base, C)])

    return k(*ins)


def _stats_call(cols):
    n = len(cols)

    def body(*refs):
        in_refs = refs[:n]
        out_ref = refs[n]
        rows = lax.broadcasted_iota(jnp.int32, (8, 128), 0)
        colsq = lax.broadcasted_iota(jnp.int32, (8, 128), 1)
        acc = jnp.zeros((8, 128), F32)
        for c, ref in enumerate(in_refs):
            f = jnp.sqrt(ref[...] + 1e-12)
            s = jnp.sum(f)
            s2 = jnp.sum(f * f)
            acc = acc + jnp.where((rows == c) & (colsq == 0), s, 0.0)
            acc = acc + jnp.where((rows == c) & (colsq == 1), s2, 0.0)
        out_ref[...] = acc

    return pl.pallas_call(
        body,
        out_shape=jax.ShapeDtypeStruct((8, 128), F32),
    )(*cols)


def _tables0_call(x0, we, be, w00a, w00b, w01a):
    def body(x_ref, we_ref, be_ref, wa_ref, wb_ref, wc_ref,
             emb_ref, a00_ref, b00_ref, a01_ref):
        e = jnp.dot(x_ref[...], we_ref[...],
                    preferred_element_type=F32) + be_ref[...]
        emb_ref[...] = e
        a00_ref[...] = jnp.dot(e, wa_ref[...], preferred_element_type=F32)
        b00_ref[...] = jnp.dot(e, wb_ref[...], preferred_element_type=F32)
        a01_ref[...] = jnp.dot(e, wc_ref[...], preferred_element_type=F32)

    blk = 2000
    wspec = pl.BlockSpec((128, 128), lambda i: (0, 0))
    bspec = pl.BlockSpec((1, 128), lambda i: (0, 0))
    rspec = pl.BlockSpec((blk, 128), lambda i: (i, 0))
    return pl.pallas_call(
        body,
        grid=(N0 // blk,),
        in_specs=[rspec, wspec, bspec, wspec, wspec, wspec],
        out_specs=[rspec] * 4,
        out_shape=[jax.ShapeDtypeStruct((N0, H), F32)] * 4,
    )(x0, we, be, w00a, w00b, w01a)


def _tables1_call(x1, msq, sv, we, be, w01b, w11a, w11b, w01dr, w11ds, w11dr):
    def body(x_ref, m_ref, sv_ref, we_ref, be_ref, wb_ref, wa1_ref, wb1_ref,
             r01_ref, rs1_ref, rr1_ref,
             emb_ref, b01_ref, a11_ref, b11_ref):
        e = jnp.dot(x_ref[...], we_ref[...],
                    preferred_element_type=F32) + be_ref[...]
        emb_ref[...] = e
        d1 = jnp.sqrt(m_ref[...] + 1e-12)
        b01_ref[...] = (jnp.dot(e, wb_ref[...], preferred_element_type=F32)
                        + (d1 - sv_ref[0, 0]) * sv_ref[0, 1] * r01_ref[...])
        a11_ref[...] = (jnp.dot(e, wa1_ref[...], preferred_element_type=F32)
                        + (d1 - sv_ref[0, 2]) * sv_ref[0, 3] * rs1_ref[...])
        b11_ref[...] = (jnp.dot(e, wb1_ref[...], preferred_element_type=F32)
                        + (d1 - sv_ref[0, 4]) * sv_ref[0, 5] * rr1_ref[...])

    blk = 2000
    wspec = pl.BlockSpec((128, 128), lambda i: (0, 0))
    vspec = pl.BlockSpec((1, 128), lambda i: (0, 0))
    rspec = pl.BlockSpec((blk, 128), lambda i: (i, 0))
    mspec = pl.BlockSpec((blk, 1), lambda i: (i, 0))
    return pl.pallas_call(
        body,
        grid=(N1 // blk,),
        in_specs=[rspec, mspec, vspec, wspec, vspec, wspec, wspec, wspec,
                  vspec, vspec, vspec],
        out_specs=[rspec] * 4,
        out_shape=[jax.ShapeDtypeStruct((N1, H), F32)] * 4,
    )(x1, msq, sv, we, be, w01b, w11a, w11b, w01dr, w11ds, w11dr)


def _edge_gather_call(ep, a_tab, b_tab, s_idx, r_idx):
    epw = ep // NW
    K = 256
    nb = epw // K

    @functools.partial(
        pl.kernel,
        out_type=jax.ShapeDtypeStruct((ep, H), F32),
        mesh=_vsmesh(),
        compiler_params=pltpu.CompilerParams(needs_layout_passes=False, use_tc_tiling_on_sc=False),
        scratch_types=[
            pltpu.VMEM((K,), jnp.int32),
            pltpu.VMEM((K,), jnp.int32),
            pltpu.VMEM((K, H), F32),
            pltpu.VMEM((K, H), F32),
            pltpu.SemaphoreType.DMA,
        ],
    )
    def k(a_h, b_h, s_h, r_h, out_h, sv, rv, bufa, bufb, sem):
        wid = lax.axis_index("s") * NC + lax.axis_index("c")
        wbase = wid * epw

        def chunk(t, carry):
            base = wbase + t * K
            pltpu.sync_copy(s_h.at[pl.ds(base, K)], sv)
            pltpu.sync_copy(r_h.at[pl.ds(base, K)], rv)
            cps = []
            for kk in range(K // 128):
                sl = pl.ds(kk * 128, 128)
                cps.append(pltpu.async_copy(
                    a_h.at[sv.at[sl]], bufa.at[sl], sem))
                cps.append(pltpu.async_copy(
                    b_h.at[rv.at[sl]], bufb.at[sl], sem))
            for cp in cps:
                cp.wait()

            def addrow(i, c2):
                for c8 in range(8):
                    sl2 = pl.ds(c8 * 16, 16)
                    bufa[i, sl2] = bufa[i, sl2] + bufb[i, sl2]
                return c2

            lax.fori_loop(0, K, addrow, 0)
            pltpu.sync_copy(bufa, out_h.at[pl.ds(base, K)])
            return carry

        lax.fori_loop(0, nb, chunk, 0)

    return k(a_tab, b_tab, s_idx, r_idx)


def _edge_mlp_call(ep, e_real, h_sum, dsq_col, sv, wd, b1, w2, b2, wg):
    blk = 1024

    def body(h_ref, d_ref, sv_ref, wd_ref, b1_ref, w2_ref, b2_ref, wg_ref,
             lo_ref, hi_ref):
        i = pl.program_id(0)
        dn = (jnp.sqrt(d_ref[...] + 1e-12) - sv_ref[0, 0]) * sv_ref[0, 1]
        hp = h_ref[...] + dn * wd_ref[...] + b1_ref[...]
        hp = hp * jax.nn.sigmoid(hp)
        m = jnp.dot(hp, w2_ref[...], preferred_element_type=F32) + b2_ref[...]
        m = m * jax.nn.sigmoid(m)
        g = jax.nn.sigmoid(
            jnp.dot(m, wg_ref[...], preferred_element_type=F32) + sv_ref[0, 2])
        mg = m * g
        rows = i * blk + lax.broadcasted_iota(jnp.int32, (blk, 1), 0)
        mg = jnp.where(rows < e_real, mg, 0.0)
        lo_ref[...] = mg[:, :64]
        hi_ref[...] = mg[:, 64:]

    wspec = pl.BlockSpec((128, 128), lambda i: (0, 0))
    vspec = pl.BlockSpec((1, 128), lambda i: (0, 0))
    return pl.pallas_call(
        body,
        grid=(ep // blk,),
        in_specs=[pl.BlockSpec((blk, 128), lambda i: (i, 0)),
                  pl.BlockSpec((blk, 1), lambda i: (i, 0)),
                  vspec, vspec, vspec, wspec, vspec,
                  pl.BlockSpec((128, 1), lambda i: (0, 0))],
        out_specs=[pl.BlockSpec((blk, 64), lambda i: (i, 0))] * 2,
        out_shape=[jax.ShapeDtypeStruct((ep, 64), F32)] * 2,
    )(h_sum, dsq_col, sv, wd, b1, w2, b2, wg)


N0T = 632
N1T = 1256
N0SH = N0T * NS
N1SH = N1T * NS
KSC = 128


def _scatter_call(m00lo, m00hi, m01lo, m01hi, m11lo, m11hi,
                  r00, r01, r11, zer):
    @functools.partial(
        pl.kernel,
        out_type=[jax.ShapeDtypeStruct((N0SH, 64), F32)] * 2
        + [jax.ShapeDtypeStruct((N1SH, 64), F32)] * 2,
        mesh=_vsmesh(),
        compiler_params=pltpu.CompilerParams(needs_layout_passes=False, use_tc_tiling_on_sc=False),
        scratch_types=[
            pltpu.VMEM_SHARED((N0SH, 64), F32),
            pltpu.VMEM_SHARED((N1SH, 64), F32),
            pltpu.VMEM((KSC,), jnp.int32),
            pltpu.VMEM((KSC, 64), F32),
        ],
    )
    def k(m00lo_h, m00hi_h, m01lo_h, m01hi_h, m11lo_h, m11hi_h,
          r00_h, r01_h, r11_h, zer_h,
          a0lo_h, a0hi_h, a1lo_h, a1hi_h,
          a0sh, a1sh, idxv, mbuf):
        c = lax.axis_index("c")
        s = lax.axis_index("s")
        pltpu.sync_copy(zer_h.at[pl.ds(0, N0T)],
                        a0sh.at[pl.ds(s * N0T, N0T)])
        pltpu.sync_copy(zer_h, a1sh.at[pl.ds(s * N1T, N1T)])
        plsc.subcore_barrier()

        def run(m_h, r_h, ep, ash):
            ept = ep // NS

            def chunk(t, carry):
                base = s * ept + t * KSC
                pltpu.sync_copy(r_h.at[pl.ds(base, KSC)], idxv)
                pltpu.sync_copy(m_h.at[pl.ds(base, KSC)], mbuf)
                pltpu.sync_copy(mbuf, ash.at[idxv], add=True)
                return carry

            lax.fori_loop(0, ept // KSC, chunk, 0)

        @pl.when(c == 0)
        def _():
            run(m00lo_h, r00_h, E00P, a0sh)
            run(m01lo_h, r01_h, E01P, a1sh)
            run(m11lo_h, r11_h, E11P, a1sh)

        @pl.when(c == 1)
        def _():
            run(m00hi_h, r00_h, E00P, a0sh)
            run(m01hi_h, r01_h, E01P, a1sh)
            run(m11hi_h, r11_h, E11P, a1sh)

        plsc.subcore_barrier()

        @pl.when(c == 0)
        def _():
            pltpu.sync_copy(a0sh.at[pl.ds(s * N0T, N0T)],
                            a0lo_h.at[pl.ds(s * N0T, N0T)])
            pltpu.sync_copy(a1sh.at[pl.ds(s * N1T, N1T)],
                            a1lo_h.at[pl.ds(s * N1T, N1T)])

        @pl.when(c == 1)
        def _():
            pltpu.sync_copy(a0sh.at[pl.ds(s * N0T, N0T)],
                            a0hi_h.at[pl.ds(s * N0T, N0T)])
            pltpu.sync_copy(a1sh.at[pl.ds(s * N1T, N1T)],
                            a1hi_h.at[pl.ds(s * N1T, N1T)])

    return k(m00lo, m00hi, m01lo, m01hi, m11lo, m11hi,
             r00, r01, r11, zer)


def _update_call(n, emb, agglo, agghi, wux, wualo, wuahi, bu, wp, bvec):
    blk = 2000

    def body(e_ref, lo_ref, hi_ref, wux_ref, wlo_ref, whi_ref, bu_ref,
             wp_ref, bv_ref, out_ref):
        e = e_ref[...]
        xn = (e + jnp.dot(e, wux_ref[...], preferred_element_type=F32)
              + jnp.dot(lo_ref[...], wlo_ref[...], preferred_element_type=F32)
              + jnp.dot(hi_ref[...], whi_ref[...], preferred_element_type=F32)
              + bu_ref[...])
        out_ref[...] = (jnp.dot(xn, wp_ref[...], preferred_element_type=F32)
                        + bv_ref[0, 0])

    wspec = pl.BlockSpec((128, 128), lambda i: (0, 0))
    hspec = pl.BlockSpec((64, 128), lambda i: (0, 0))
    vspec = pl.BlockSpec((1, 128), lambda i: (0, 0))
    return pl.pallas_call(
        body,
        grid=(n // blk,),
        in_specs=[pl.BlockSpec((blk, 128), lambda i: (i, 0)),
                  pl.BlockSpec((blk, 64), lambda i: (i, 0)),
                  pl.BlockSpec((blk, 64), lambda i: (i, 0)),
                  wspec, hspec, hspec, vspec,
                  pl.BlockSpec((128, 1), lambda i: (0, 0)), vspec],
        out_specs=pl.BlockSpec((blk, 1), lambda i: (i, 0)),
        out_shape=jax.ShapeDtypeStruct((n, 1), F32),
    )(emb, agglo, agghi, wux, wualo, wuahi, bu, wp, bvec)


def _pad1(x, n, val):
    return jnp.concatenate(
        [x, jnp.full((n - x.shape[0],), val, dtype=x.dtype)])


def _stat(sums, row, count):
    mu = sums[row, 0] / count
    var = sums[row, 1] / count - mu * mu
    isig = lax.rsqrt(var + 1e-5)
    return mu, isig


def kernel(pos, x_0, x_1, cell_1, adj_0_0, adj_0_1, adj_1_1, params):
    posx, posy, posz = pos[:, 0], pos[:, 1], pos[:, 2]
    cpad = jnp.pad(cell_1, ((0, N1P - N1), (0, 0)))
    c0, c1_, c2, c3 = (cpad[:, k] for k in range(4))

    s00 = _pad1(adj_0_0[0], E00P, 0)
    r00 = _pad1(adj_0_0[1], E00P, 0)
    s01 = _pad1(adj_0_1[0], E01P, 0)
    r01 = _pad1(adj_0_1[1], E01P, 0)
    s11 = _pad1(adj_1_1[0], E11P, 0)
    r11 = _pad1(adj_1_1[1], E11P, 0)

    ccx, ccy, ccz, cmsq = _geom_call(posx, posy, posz, c0, c1_, c2, c3)
    tabs0 = (posx, posy, posz)
    tabs1 = (ccx[:N1], ccy[:N1], ccz[:N1])
    msq1 = cmsq[:N1]

    (d00,) = _edge_feat_call(E00P, 0, 0, tabs0, tabs1, msq1, s00, r00)
    d01, dr01 = _edge_feat_call(E01P, 0, 1, tabs0, tabs1, msq1, s01, r01)
    d11, ds11, dr11 = _edge_feat_call(E11P, 1, 1, tabs0, tabs1, msq1, s11, r11)

    st00 = _stats_call([d00[:E00].reshape(-1, 128)])
    st01 = _stats_call([d01[:E01].reshape(-1, 128),
                        dr01[:E01].reshape(-1, 128)])
    st11 = _stats_call([d11[:E11].reshape(-1, 128),
                        ds11[:E11].reshape(-1, 128),
                        dr11[:E11].reshape(-1, 128)])

    mu00d, is00d = _stat(st00, 0, E00)
    mu01d, is01d = _stat(st01, 0, E01)
    mu01r, is01r = _stat(st01, 1, E01)
    mu11d, is11d = _stat(st11, 0, E11)
    mu11s, is11s = _stat(st11, 1, E11)
    mu11r, is11r = _stat(st11, 2, E11)

    p = params
    w1_00, w1_01, w1_11 = p["W1_0_0"], p["W1_0_1"], p["W1_1_1"]

    emb0, a00, b00, a01 = _tables0_call(
        x_0, p["W_emb_0"], p["b_emb_0"].reshape(1, H),
        w1_00[:H], w1_00[H:2 * H], w1_01[:H])
    sv1 = jnp.stack([mu01r, is01r, mu11s, is11s, mu11r, is11r])
    sv1 = jnp.pad(sv1, (0, 128 - 6)).reshape(1, 128)
    emb1, b01, a11, b11 = _tables1_call(
        x_1, msq1.reshape(N1, 1), sv1,
        p["W_emb_1"], p["b_emb_1"].reshape(1, H),
        w1_01[H:2 * H], w1_11[:H], w1_11[H:2 * H],
        w1_01[2 * H + 2].reshape(1, H),
        w1_11[2 * H + 1].reshape(1, H),
        w1_11[2 * H + 2].reshape(1, H))

    h00 = _edge_gather_call(E00P, a00, b00, s00, r00)
    h01 = _edge_gather_call(E01P, a01, b01, s01, r01)
    h11 = _edge_gather_call(E11P, a11, b11, s11, r11)

    def mlp(ep, e_real, h, dsq, mu, isig, a):
        sv = jnp.stack([mu, isig, p[f"bg_{a}"][0]])
        sv = jnp.pad(sv, (0, 125)).reshape(1, 128)
        return _edge_mlp_call(
            ep, e_real, h, dsq.reshape(ep, 1), sv,
            p[f"W1_{a}"][2 * H].reshape(1, H),
            p[f"b1_{a}"].reshape(1, H),
            p[f"W2_{a}"], p[f"b2_{a}"].reshape(1, H),
            p[f"Wg_{a}"])

    m00lo, m00hi = mlp(E00P, E00, h00, d00, mu00d, is00d, "0_0")
    m01lo, m01hi = mlp(E01P, E01, h01, d01, mu01d, is01d, "0_1")
    m11lo, m11hi = mlp(E11P, E11, h11, d11, mu11d, is11d, "1_1")

    zer = jnp.zeros((N1T, 64), F32)
    a0lo, a0hi, a1lo, a1hi = _scatter_call(
        m00lo, m00hi, m01lo, m01hi, m11lo, m11hi, r00, r01, r11, zer)
    a0lo, a0hi = a0lo[:N0], a0hi[:N0]
    a1lo, a1hi = a1lo[:N1], a1hi[:N1]

    wu0, wu1 = p["W_upd_0"], p["W_upd_1"]
    out0 = _update_call(
        N0, emb0, a0lo, a0hi, wu0[:H], wu0[H:H + 64], wu0[H + 64:],
        p["b_upd_0"].reshape(1, H), p["W_pre_0"],
        jnp.pad(p["b_pre_0"], (0, 127)).reshape(1, 128))
    out1 = _update_call(
        N1, emb1, a1lo, a1hi, wu1[:H], wu1[H:H + 64], wu1[H + 64:],
        p["b_upd_1"].reshape(1, H), p["W_pre_1"],
        jnp.pad(p["b_pre_1"], (0, 127)).reshape(1, 128))
    return out0, out1


# speedup vs baseline: 5.6727x; 1.1969x over previous
import functools

import jax
import jax.numpy as jnp
from jax import lax
from jax.experimental import pallas as pl
from jax.experimental.pallas import tpu as pltpu
from jax.experimental.pallas import tpu_sc as plsc

N0 = 10000
N1 = 20000
H = 128
E00 = 320000
E01 = 160000
E11 = 160000
E00P = 327680
E01P = 163840
E11P = 163840
NC = 2
NS = 16
NW = NC * NS
N1P = 20480
F32 = jnp.float32


def _vsmesh():
    return plsc.VectorSubcoreMesh(
        core_axis_name="c", subcore_axis_name="s", num_cores=NC, num_subcores=NS)


def _geom_call(px, py, pz, c0, c1, c2, c3):
    cpw = N1P // NW

    @functools.partial(
        pl.kernel,
        out_type=[jax.ShapeDtypeStruct((N1P,), F32)] * 4,
        mesh=_vsmesh(),
        compiler_params=pltpu.CompilerParams(needs_layout_passes=False, use_tc_tiling_on_sc=False),
        scratch_types=(
            [pltpu.VMEM((N0,), F32)] * 3
            + [pltpu.VMEM((cpw,), jnp.int32)] * 4
            + [pltpu.VMEM((cpw,), F32)] * 4
        ),
    )
    def k(px_h, py_h, pz_h, c0_h, c1_h, c2_h, c3_h,
          ocx_h, ocy_h, ocz_h, om_h,
          pxv, pyv, pzv, i0, i1, i2, i3, ox, oy, oz, om):
        wid = lax.axis_index("s") * NC + lax.axis_index("c")
        base = wid * cpw
        pltpu.sync_copy(px_h, pxv)
        pltpu.sync_copy(py_h, pyv)
        pltpu.sync_copy(pz_h, pzv)
        pltpu.sync_copy(c0_h.at[pl.ds(base, cpw)], i0)
        pltpu.sync_copy(c1_h.at[pl.ds(base, cpw)], i1)
        pltpu.sync_copy(c2_h.at[pl.ds(base, cpw)], i2)
        pltpu.sync_copy(c3_h.at[pl.ds(base, cpw)], i3)

        def body(t, carry):
            o = pl.multiple_of(t * 16, 16)
            idx = [i0[pl.ds(o, 16)], i1[pl.ds(o, 16)],
                   i2[pl.ds(o, 16)], i3[pl.ds(o, 16)]]
            xs = [plsc.load_gather(pxv, [a]) for a in idx]
            ys = [plsc.load_gather(pyv, [a]) for a in idx]
            zs = [plsc.load_gather(pzv, [a]) for a in idx]
            ox[pl.ds(o, 16)] = (xs[0] + xs[1] + xs[2] + xs[3]) * 0.25
            oy[pl.ds(o, 16)] = (ys[0] + ys[1] + ys[2] + ys[3]) * 0.25
            oz[pl.ds(o, 16)] = (zs[0] + zs[1] + zs[2] + zs[3]) * 0.25
            m = jnp.zeros((16,), F32)
            for p in range(4):
                for q in range(p + 1, 4):
                    dx = xs[p] - xs[q]
                    dy = ys[p] - ys[q]
                    dz = zs[p] - zs[q]
                    m = jnp.maximum(m, dx * dx + dy * dy + dz * dz)
            om[pl.ds(o, 16)] = m
            return carry

        lax.fori_loop(0, cpw // 16, body, 0)
        pltpu.sync_copy(ox, ocx_h.at[pl.ds(base, cpw)])
        pltpu.sync_copy(oy, ocy_h.at[pl.ds(base, cpw)])
        pltpu.sync_copy(oz, ocz_h.at[pl.ds(base, cpw)])
        pltpu.sync_copy(om, om_h.at[pl.ds(base, cpw)])

    return k(px, py, pz, c0, c1, c2, c3)


def _edge_feat_call(ep, src_dim, dst_dim, tabs0, tabs1, m1, s_idx, r_idx):
    epw = ep // NW
    C = 1280
    nchunks = epw // C
    n_out = 1 + (src_dim == 1) + (dst_dim == 1)

    src_tabs = tabs0 if src_dim == 0 else tabs1
    dst_tabs = tabs0 if dst_dim == 0 else tabs1
    n_src = N0 if src_dim == 0 else N1
    n_dst = N0 if dst_dim == 0 else N1
    same = src_dim == dst_dim

    scratch = [pltpu.VMEM((n_src,), F32)] * 3
    if not same:
        scratch += [pltpu.VMEM((n_dst,), F32)] * 3
    need_m1 = (src_dim == 1) or (dst_dim == 1)
    if need_m1:
        scratch += [pltpu.VMEM((N1,), F32)]
    scratch += [pltpu.VMEM((C,), jnp.int32)] * 2
    scratch += [pltpu.VMEM((C,), F32)] * n_out

    ins = list(src_tabs) + ([] if same else list(dst_tabs))
    if need_m1:
        ins.append(m1)
    ins += [s_idx, r_idx]

    @functools.partial(
        pl.kernel,
        out_type=[jax.ShapeDtypeStruct((ep,), F32)] * n_out,
        mesh=_vsmesh(),
        compiler_params=pltpu.CompilerParams(needs_layout_passes=False, use_tc_tiling_on_sc=False),
        scratch_types=scratch,
    )
    def k(*refs):
        pos = 0
        sx_h, sy_h, sz_h = refs[pos:pos + 3]; pos += 3
        if not same:
            dx_h, dy_h, dz_h = refs[pos:pos + 3]; pos += 3
        else:
            dx_h, dy_h, dz_h = sx_h, sy_h, sz_h
        if need_m1:
            m1_h = refs[pos]; pos += 1
        s_h, r_h = refs[pos:pos + 2]; pos += 2
        out_hs = refs[pos:pos + n_out]; pos += n_out
        sxv, syv, szv = refs[pos:pos + 3]; pos += 3
        if not same:
            dxv, dyv, dzv = refs[pos:pos + 3]; pos += 3
        else:
            dxv, dyv, dzv = sxv, syv, szv
        if need_m1:
            m1v = refs[pos]; pos += 1
        sv, rv = refs[pos:pos + 2]; pos += 2
        obufs = refs[pos:pos + n_out]

        wid = lax.axis_index("s") * NC + lax.axis_index("c")
        wbase = wid * epw
        pltpu.sync_copy(sx_h, sxv)
        pltpu.sync_copy(sy_h, syv)
        pltpu.sync_copy(sz_h, szv)
        if not same:
            pltpu.sync_copy(dx_h, dxv)
            pltpu.sync_copy(dy_h, dyv)
            pltpu.sync_copy(dz_h, dzv)
        if need_m1:
            pltpu.sync_copy(m1_h, m1v)

        for cc in range(nchunks):
            base = wbase + cc * C
            pltpu.sync_copy(s_h.at[pl.ds(base, C)], sv)
            pltpu.sync_copy(r_h.at[pl.ds(base, C)], rv)

            def body(t, carry):
                o = pl.multiple_of(t * 16, 16)
                si = sv[pl.ds(o, 16)]
                ri = rv[pl.ds(o, 16)]
                ax = plsc.load_gather(sxv, [si])
                ay = plsc.load_gather(syv, [si])
                az = plsc.load_gather(szv, [si])
                bx = plsc.load_gather(dxv, [ri])
                by = plsc.load_gather(dyv, [ri])
                bz = plsc.load_gather(dzv, [ri])
                ex = ax - bx
                ey = ay - by
                ez = az - bz
                ob = 0
                obufs[ob][pl.ds(o, 16)] = ex * ex + ey * ey + ez * ez
                ob += 1
                if src_dim == 1:
                    obufs[ob][pl.ds(o, 16)] = plsc.load_gather(m1v, [si])
                    ob += 1
                if dst_dim == 1:
                    obufs[ob][pl.ds(o, 16)] = plsc.load_gather(m1v, [ri])
                return carry

            lax.fori_loop(0, C // 16, body, 0)
            for b, oh in zip(obufs, out_hs):
                pltpu.sync_copy(b, oh.at[pl.ds(base, C)])

    return k(*ins)


def _stats_call(cols):
    n = len(cols)

    def body(*refs):
        in_refs = refs[:n]
        out_ref = refs[n]
        rows = lax.broadcasted_iota(jnp.int32, (8, 128), 0)
        colsq = lax.broadcasted_iota(jnp.int32, (8, 128), 1)
        acc = jnp.zeros((8, 128), F32)
        for c, ref in enumerate(in_refs):
            f = jnp.sqrt(ref[...] + 1e-12)
            s = jnp.sum(f)
            s2 = jnp.sum(f * f)
            acc = acc + jnp.where((rows == c) & (colsq == 0), s, 0.0)
            acc = acc + jnp.where((rows == c) & (colsq == 1), s2, 0.0)
        out_ref[...] = acc

    return pl.pallas_call(
        body,
        out_shape=jax.ShapeDtypeStruct((8, 128), F32),
    )(*cols)


def _tables0_call(x0, we, be, w00a, w00b, w01a):
    def body(x_ref, we_ref, be_ref, wa_ref, wb_ref, wc_ref,
             emb_ref, a00_ref, b00_ref, a01_ref):
        e = jnp.dot(x_ref[...], we_ref[...],
                    preferred_element_type=F32) + be_ref[...]
        emb_ref[...] = e
        a00_ref[...] = jnp.dot(e, wa_ref[...], preferred_element_type=F32)
        b00_ref[...] = jnp.dot(e, wb_ref[...], preferred_element_type=F32)
        a01_ref[...] = jnp.dot(e, wc_ref[...], preferred_element_type=F32)

    blk = 2000
    wspec = pl.BlockSpec((128, 128), lambda i: (0, 0))
    bspec = pl.BlockSpec((1, 128), lambda i: (0, 0))
    rspec = pl.BlockSpec((blk, 128), lambda i: (i, 0))
    return pl.pallas_call(
        body,
        grid=(N0 // blk,),
        in_specs=[rspec, wspec, bspec, wspec, wspec, wspec],
        out_specs=[rspec] * 4,
        out_shape=[jax.ShapeDtypeStruct((N0, H), F32)] * 4,
    )(x0, we, be, w00a, w00b, w01a)


def _tables1_call(x1, msq, sv, we, be, w01b, w11a, w11b, w01dr, w11ds, w11dr):
    def body(x_ref, m_ref, sv_ref, we_ref, be_ref, wb_ref, wa1_ref, wb1_ref,
             r01_ref, rs1_ref, rr1_ref,
             emb_ref, b01_ref, a11_ref, b11_ref):
        e = jnp.dot(x_ref[...], we_ref[...],
                    preferred_element_type=F32) + be_ref[...]
        emb_ref[...] = e
        d1 = jnp.sqrt(m_ref[...] + 1e-12)
        b01_ref[...] = (jnp.dot(e, wb_ref[...], preferred_element_type=F32)
                        + (d1 - sv_ref[0, 0]) * sv_ref[0, 1] * r01_ref[...])
        a11_ref[...] = (jnp.dot(e, wa1_ref[...], preferred_element_type=F32)
                        + (d1 - sv_ref[0, 2]) * sv_ref[0, 3] * rs1_ref[...])
        b11_ref[...] = (jnp.dot(e, wb1_ref[...], preferred_element_type=F32)
                        + (d1 - sv_ref[0, 4]) * sv_ref[0, 5] * rr1_ref[...])

    blk = 2000
    wspec = pl.BlockSpec((128, 128), lambda i: (0, 0))
    vspec = pl.BlockSpec((1, 128), lambda i: (0, 0))
    rspec = pl.BlockSpec((blk, 128), lambda i: (i, 0))
    mspec = pl.BlockSpec((blk, 1), lambda i: (i, 0))
    return pl.pallas_call(
        body,
        grid=(N1 // blk,),
        in_specs=[rspec, mspec, vspec, wspec, vspec, wspec, wspec, wspec,
                  vspec, vspec, vspec],
        out_specs=[rspec] * 4,
        out_shape=[jax.ShapeDtypeStruct((N1, H), F32)] * 4,
    )(x1, msq, sv, we, be, w01b, w11a, w11b, w01dr, w11ds, w11dr)


def _edge_gather_call(ep, a_tab, b_tab, s_idx, r_idx):
    epw = ep // NW
    K = 128
    nb = epw // K

    @functools.partial(
        pl.kernel,
        out_type=jax.ShapeDtypeStruct((ep, H), F32),
        mesh=_vsmesh(),
        compiler_params=pltpu.CompilerParams(needs_layout_passes=False, use_tc_tiling_on_sc=False),
        scratch_types=[
            [pltpu.VMEM((K,), jnp.int32)] * 2,
            [pltpu.VMEM((K,), jnp.int32)] * 2,
            [pltpu.VMEM((K, H), F32)] * 2,
            [pltpu.VMEM((K, H), F32)] * 2,
            [pltpu.SemaphoreType.DMA] * 2,
            [pltpu.SemaphoreType.DMA] * 2,
        ],
    )
    def k(a_h, b_h, s_h, r_h, out_h, sv, rv, bufa, bufb, gsem, osem):
        wid = lax.axis_index("s") * NC + lax.axis_index("c")
        wbase = wid * epw

        def issue(t, p):
            base = wbase + t * K
            pltpu.sync_copy(s_h.at[pl.ds(base, K)], sv[p])
            pltpu.sync_copy(r_h.at[pl.ds(base, K)], rv[p])
            pltpu.async_copy(a_h.at[sv[p]], bufa[p], gsem[p])
            pltpu.async_copy(b_h.at[rv[p]], bufb[p], gsem[p])

        def drain_gather(t, p):
            pltpu.make_async_copy(a_h.at[sv[p]], bufa[p], gsem[p]).wait()
            pltpu.make_async_copy(b_h.at[rv[p]], bufb[p], gsem[p]).wait()

        def consume(t, p):
            base = wbase + t * K

            def addrow(i, c2):
                for c8 in range(8):
                    sl2 = pl.ds(c8 * 16, 16)
                    bufa[p][i, sl2] = bufa[p][i, sl2] + bufb[p][i, sl2]
                return c2

            lax.fori_loop(0, K, addrow, 0)
            pltpu.async_copy(bufa[p], out_h.at[pl.ds(base, K)], osem[p])

        issue(0, 0)

        def pair(i, carry):
            for sub in range(2):
                t = 2 * i + sub
                nxt = 1 - sub

                @pl.when(t + 1 < nb)
                def _():
                    @pl.when(t >= 1)
                    def _():
                        pltpu.make_async_copy(
                            bufa[nxt], out_h.at[pl.ds(0, K)], osem[nxt]).wait()
                    issue(t + 1, nxt)

                drain_gather(t, sub)
                consume(t, sub)
            return carry

        lax.fori_loop(0, nb // 2, pair, 0)
        pltpu.make_async_copy(bufa[0], out_h.at[pl.ds(0, K)], osem[0]).wait()
        pltpu.make_async_copy(bufa[1], out_h.at[pl.ds(0, K)], osem[1]).wait()

    return k(a_tab, b_tab, s_idx, r_idx)


def _edge_mlp_call(ep, e_real, h_sum, dsq_col, sv, wd, b1, w2, b2, wg):
    blk = 1024

    def body(h_ref, d_ref, sv_ref, wd_ref, b1_ref, w2_ref, b2_ref, wg_ref,
             lo_ref, hi_ref):
        i = pl.program_id(0)
        dn = (jnp.sqrt(d_ref[...] + 1e-12) - sv_ref[0, 0]) * sv_ref[0, 1]
        hp = h_ref[...] + dn * wd_ref[...] + b1_ref[...]
        hp = hp * jax.nn.sigmoid(hp)
        m = jnp.dot(hp, w2_ref[...], preferred_element_type=F32) + b2_ref[...]
        m = m * jax.nn.sigmoid(m)
        g = jax.nn.sigmoid(
            jnp.dot(m, wg_ref[...], preferred_element_type=F32) + sv_ref[0, 2])
        mg = m * g
        rows = i * blk + lax.broadcasted_iota(jnp.int32, (blk, 1), 0)
        mg = jnp.where(rows < e_real, mg, 0.0)
        lo_ref[...] = mg[:, :64]
        hi_ref[...] = mg[:, 64:]

    wspec = pl.BlockSpec((128, 128), lambda i: (0, 0))
    vspec = pl.BlockSpec((1, 128), lambda i: (0, 0))
    return pl.pallas_call(
        body,
        grid=(ep // blk,),
        in_specs=[pl.BlockSpec((blk, 128), lambda i: (i, 0)),
                  pl.BlockSpec((blk, 1), lambda i: (i, 0)),
                  vspec, vspec, vspec, wspec, vspec,
                  pl.BlockSpec((128, 1), lambda i: (0, 0))],
        out_specs=[pl.BlockSpec((blk, 64), lambda i: (i, 0))] * 2,
        out_shape=[jax.ShapeDtypeStruct((ep, 64), F32)] * 2,
    )(h_sum, dsq_col, sv, wd, b1, w2, b2, wg)


N0T = 632
N1T = 1256
N0SH = N0T * NS
N1SH = N1T * NS
KSC = 128


def _scatter_call(m00lo, m00hi, m01lo, m01hi, m11lo, m11hi,
                  r00, r01, r11, zer):
    @functools.partial(
        pl.kernel,
        out_type=[jax.ShapeDtypeStruct((N0SH, 64), F32)] * 2
        + [jax.ShapeDtypeStruct((N1SH, 64), F32)] * 2,
        mesh=_vsmesh(),
        compiler_params=pltpu.CompilerParams(needs_layout_passes=False, use_tc_tiling_on_sc=False),
        scratch_types=[
            pltpu.VMEM_SHARED((N1SH, 64), F32),
            [pltpu.VMEM((KSC,), jnp.int32)] * 2,
            [pltpu.VMEM((KSC, 64), F32)] * 2,
            [pltpu.SemaphoreType.DMA] * 2,
        ],
    )
    def k(m00lo_h, m00hi_h, m01lo_h, m01hi_h, m11lo_h, m11hi_h,
          r00_h, r01_h, r11_h, zer_h,
          a0lo_h, a0hi_h, a1lo_h, a1hi_h,
          ash, idxv, mbuf, sem):
        c = lax.axis_index("c")
        s = lax.axis_index("s")

        def run(m_h, r_h, ep, ash):
            ept = ep // NS

            def issue(t, p):
                base = s * ept + t * KSC
                pltpu.async_copy(r_h.at[pl.ds(base, KSC)], idxv[p], sem[p])
                pltpu.async_copy(m_h.at[pl.ds(base, KSC)], mbuf[p], sem[p])

            def drain(t, p):
                base = s * ept + t * KSC
                pltpu.make_async_copy(
                    r_h.at[pl.ds(base, KSC)], idxv[p], sem[p]).wait()
                pltpu.make_async_copy(
                    m_h.at[pl.ds(base, KSC)], mbuf[p], sem[p]).wait()

            nb = ept // KSC
            issue(0, 0)

            def pair(i, carry):
                for sub in range(2):
                    t = 2 * i + sub

                    @pl.when(t + 1 < nb)
                    def _():
                        issue(t + 1, 1 - sub)

                    drain(t, sub)
                    pltpu.sync_copy(mbuf[sub], ash.at[idxv[sub]], add=True)
                return carry

            lax.fori_loop(0, nb // 2, pair, 0)

        pltpu.sync_copy(zer_h.at[pl.ds(0, N0T)], ash.at[pl.ds(s * N0T, N0T)])
        plsc.subcore_barrier()

        @pl.when(c == 0)
        def _():
            run(m00lo_h, r00_h, E00P, ash)

        @pl.when(c == 1)
        def _():
            run(m00hi_h, r00_h, E00P, ash)

        plsc.subcore_barrier()

        @pl.when(c == 0)
        def _():
            pltpu.sync_copy(ash.at[pl.ds(s * N0T, N0T)],
                            a0lo_h.at[pl.ds(s * N0T, N0T)])

        @pl.when(c == 1)
        def _():
            pltpu.sync_copy(ash.at[pl.ds(s * N0T, N0T)],
                            a0hi_h.at[pl.ds(s * N0T, N0T)])

        plsc.subcore_barrier()

        pltpu.sync_copy(zer_h, ash.at[pl.ds(s * N1T, N1T)])
        plsc.subcore_barrier()

        @pl.when(c == 0)
        def _():
            run(m01lo_h, r01_h, E01P, ash)
            run(m11lo_h, r11_h, E11P, ash)

        @pl.when(c == 1)
        def _():
            run(m01hi_h, r01_h, E01P, ash)
            run(m11hi_h, r11_h, E11P, ash)

        plsc.subcore_barrier()

        @pl.when(c == 0)
        def _():
            pltpu.sync_copy(ash.at[pl.ds(s * N1T, N1T)],
                            a1lo_h.at[pl.ds(s * N1T, N1T)])

        @pl.when(c == 1)
        def _():
            pltpu.sync_copy(ash.at[pl.ds(s * N1T, N1T)],
                            a1hi_h.at[pl.ds(s * N1T, N1T)])

    return k(m00lo, m00hi, m01lo, m01hi, m11lo, m11hi,
             r00, r01, r11, zer)


def _update_call(n, emb, agglo, agghi, wux, wualo, wuahi, bu, wp, bvec):
    blk = 2000

    def body(e_ref, lo_ref, hi_ref, wux_ref, wlo_ref, whi_ref, bu_ref,
             wp_ref, bv_ref, out_ref):
        e = e_ref[...]
        xn = (e + jnp.dot(e, wux_ref[...], preferred_element_type=F32)
              + jnp.dot(lo_ref[...], wlo_ref[...], preferred_element_type=F32)
              + jnp.dot(hi_ref[...], whi_ref[...], preferred_element_type=F32)
              + bu_ref[...])
        out_ref[...] = (jnp.dot(xn, wp_ref[...], preferred_element_type=F32)
                        + bv_ref[0, 0])

    wspec = pl.BlockSpec((128, 128), lambda i: (0, 0))
    hspec = pl.BlockSpec((64, 128), lambda i: (0, 0))
    vspec = pl.BlockSpec((1, 128), lambda i: (0, 0))
    return pl.pallas_call(
        body,
        grid=(n // blk,),
        in_specs=[pl.BlockSpec((blk, 128), lambda i: (i, 0)),
                  pl.BlockSpec((blk, 64), lambda i: (i, 0)),
                  pl.BlockSpec((blk, 64), lambda i: (i, 0)),
                  wspec, hspec, hspec, vspec,
                  pl.BlockSpec((128, 1), lambda i: (0, 0)), vspec],
        out_specs=pl.BlockSpec((blk, 1), lambda i: (i, 0)),
        out_shape=jax.ShapeDtypeStruct((n, 1), F32),
    )(emb, agglo, agghi, wux, wualo, wuahi, bu, wp, bvec)


def _pad1(x, n, val):
    return jnp.concatenate(
        [x, jnp.full((n - x.shape[0],), val, dtype=x.dtype)])


def _stat(sums, row, count):
    mu = sums[row, 0] / count
    var = sums[row, 1] / count - mu * mu
    isig = lax.rsqrt(var + 1e-5)
    return mu, isig


def kernel(pos, x_0, x_1, cell_1, adj_0_0, adj_0_1, adj_1_1, params):
    posx, posy, posz = pos[:, 0], pos[:, 1], pos[:, 2]
    cpad = jnp.pad(cell_1, ((0, N1P - N1), (0, 0)))
    c0, c1_, c2, c3 = (cpad[:, k] for k in range(4))

    s00 = _pad1(adj_0_0[0], E00P, 0)
    r00 = _pad1(adj_0_0[1], E00P, 0)
    s01 = _pad1(adj_0_1[0], E01P, 0)
    r01 = _pad1(adj_0_1[1], E01P, 0)
    s11 = _pad1(adj_1_1[0], E11P, 0)
    r11 = _pad1(adj_1_1[1], E11P, 0)

    ccx, ccy, ccz, cmsq = _geom_call(posx, posy, posz, c0, c1_, c2, c3)
    tabs0 = (posx, posy, posz)
    tabs1 = (ccx[:N1], ccy[:N1], ccz[:N1])
    msq1 = cmsq[:N1]

    (d00,) = _edge_feat_call(E00P, 0, 0, tabs0, tabs1, msq1, s00, r00)
    d01, dr01 = _edge_feat_call(E01P, 0, 1, tabs0, tabs1, msq1, s01, r01)
    d11, ds11, dr11 = _edge_feat_call(E11P, 1, 1, tabs0, tabs1, msq1, s11, r11)

    st00 = _stats_call([d00[:E00].reshape(-1, 128)])
    st01 = _stats_call([d01[:E01].reshape(-1, 128),
                        dr01[:E01].reshape(-1, 128)])
    st11 = _stats_call([d11[:E11].reshape(-1, 128),
                        ds11[:E11].reshape(-1, 128),
                        dr11[:E11].reshape(-1, 128)])

    mu00d, is00d = _stat(st00, 0, E00)
    mu01d, is01d = _stat(st01, 0, E01)
    mu01r, is01r = _stat(st01, 1, E01)
    mu11d, is11d = _stat(st11, 0, E11)
    mu11s, is11s = _stat(st11, 1, E11)
    mu11r, is11r = _stat(st11, 2, E11)

    p = params
    w1_00, w1_01, w1_11 = p["W1_0_0"], p["W1_0_1"], p["W1_1_1"]

    emb0, a00, b00, a01 = _tables0_call(
        x_0, p["W_emb_0"], p["b_emb_0"].reshape(1, H),
        w1_00[:H], w1_00[H:2 * H], w1_01[:H])
    sv1 = jnp.stack([mu01r, is01r, mu11s, is11s, mu11r, is11r])
    sv1 = jnp.pad(sv1, (0, 128 - 6)).reshape(1, 128)
    emb1, b01, a11, b11 = _tables1_call(
        x_1, msq1.reshape(N1, 1), sv1,
        p["W_emb_1"], p["b_emb_1"].reshape(1, H),
        w1_01[H:2 * H], w1_11[:H], w1_11[H:2 * H],
        w1_01[2 * H + 2].reshape(1, H),
        w1_11[2 * H + 1].reshape(1, H),
        w1_11[2 * H + 2].reshape(1, H))

    h00 = _edge_gather_call(E00P, a00, b00, s00, r00)
    h01 = _edge_gather_call(E01P, a01, b01, s01, r01)
    h11 = _edge_gather_call(E11P, a11, b11, s11, r11)

    def mlp(ep, e_real, h, dsq, mu, isig, a):
        sv = jnp.stack([mu, isig, p[f"bg_{a}"][0]])
        sv = jnp.pad(sv, (0, 125)).reshape(1, 128)
        return _edge_mlp_call(
            ep, e_real, h, dsq.reshape(ep, 1), sv,
            p[f"W1_{a}"][2 * H].reshape(1, H),
            p[f"b1_{a}"].reshape(1, H),
            p[f"W2_{a}"], p[f"b2_{a}"].reshape(1, H),
            p[f"Wg_{a}"])

    m00lo, m00hi = mlp(E00P, E00, h00, d00, mu00d, is00d, "0_0")
    m01lo, m01hi = mlp(E01P, E01, h01, d01, mu01d, is01d, "0_1")
    m11lo, m11hi = mlp(E11P, E11, h11, d11, mu11d, is11d, "1_1")

    zer = jnp.zeros((N1T, 64), F32)
    a0lo, a0hi, a1lo, a1hi = _scatter_call(
        m00lo, m00hi, m01lo, m01hi, m11lo, m11hi, r00, r01, r11, zer)
    a0lo, a0hi = a0lo[:N0], a0hi[:N0]
    a1lo, a1hi = a1lo[:N1], a1hi[:N1]

    wu0, wu1 = p["W_upd_0"], p["W_upd_1"]
    out0 = _update_call(
        N0, emb0, a0lo, a0hi, wu0[:H], wu0[H:H + 64], wu0[H + 64:],
        p["b_upd_0"].reshape(1, H), p["W_pre_0"],
        jnp.pad(p["b_pre_0"], (0, 127)).reshape(1, 128))
    out1 = _update_call(
        N1, emb1, a1lo, a1hi, wu1[:H], wu1[H:H + 64], wu1[H + 64:],
        p["b_upd_1"].reshape(1, H), p["W_pre_1"],
        jnp.pad(p["b_pre_1"], (0, 127)).reshape(1, 128))
    return out0, out1


# speedup vs baseline: 5.7253x; 1.0093x over previous
import functools

import jax
import jax.numpy as jnp
from jax import lax
from jax.experimental import pallas as pl
from jax.experimental.pallas import tpu as pltpu
from jax.experimental.pallas import tpu_sc as plsc

N0 = 10000
N1 = 20000
H = 128
E00 = 320000
E01 = 160000
E11 = 160000
E00P = 327680
E01P = 163840
E11P = 163840
NC = 2
NS = 16
NW = NC * NS
N1P = 20480
F32 = jnp.float32


def _vsmesh():
    return plsc.VectorSubcoreMesh(
        core_axis_name="c", subcore_axis_name="s", num_cores=NC, num_subcores=NS)


def _geom_call(px, py, pz, c0, c1, c2, c3):
    cpw = N1P // NW

    @functools.partial(
        pl.kernel,
        out_type=[jax.ShapeDtypeStruct((N1P,), F32)] * 4,
        mesh=_vsmesh(),
        compiler_params=pltpu.CompilerParams(needs_layout_passes=False, use_tc_tiling_on_sc=False),
        scratch_types=(
            [pltpu.VMEM((N0,), F32)] * 3
            + [pltpu.VMEM((cpw,), jnp.int32)] * 4
            + [pltpu.VMEM((cpw,), F32)] * 4
        ),
    )
    def k(px_h, py_h, pz_h, c0_h, c1_h, c2_h, c3_h,
          ocx_h, ocy_h, ocz_h, om_h,
          pxv, pyv, pzv, i0, i1, i2, i3, ox, oy, oz, om):
        wid = lax.axis_index("s") * NC + lax.axis_index("c")
        base = wid * cpw
        pltpu.sync_copy(px_h, pxv)
        pltpu.sync_copy(py_h, pyv)
        pltpu.sync_copy(pz_h, pzv)
        pltpu.sync_copy(c0_h.at[pl.ds(base, cpw)], i0)
        pltpu.sync_copy(c1_h.at[pl.ds(base, cpw)], i1)
        pltpu.sync_copy(c2_h.at[pl.ds(base, cpw)], i2)
        pltpu.sync_copy(c3_h.at[pl.ds(base, cpw)], i3)

        def body(t, carry):
            o = pl.multiple_of(t * 16, 16)
            idx = [i0[pl.ds(o, 16)], i1[pl.ds(o, 16)],
                   i2[pl.ds(o, 16)], i3[pl.ds(o, 16)]]
            xs = [plsc.load_gather(pxv, [a]) for a in idx]
            ys = [plsc.load_gather(pyv, [a]) for a in idx]
            zs = [plsc.load_gather(pzv, [a]) for a in idx]
            ox[pl.ds(o, 16)] = (xs[0] + xs[1] + xs[2] + xs[3]) * 0.25
            oy[pl.ds(o, 16)] = (ys[0] + ys[1] + ys[2] + ys[3]) * 0.25
            oz[pl.ds(o, 16)] = (zs[0] + zs[1] + zs[2] + zs[3]) * 0.25
            m = jnp.zeros((16,), F32)
            for p in range(4):
                for q in range(p + 1, 4):
                    dx = xs[p] - xs[q]
                    dy = ys[p] - ys[q]
                    dz = zs[p] - zs[q]
                    m = jnp.maximum(m, dx * dx + dy * dy + dz * dz)
            om[pl.ds(o, 16)] = m
            return carry

        lax.fori_loop(0, cpw // 16, body, 0)
        pltpu.sync_copy(ox, ocx_h.at[pl.ds(base, cpw)])
        pltpu.sync_copy(oy, ocy_h.at[pl.ds(base, cpw)])
        pltpu.sync_copy(oz, ocz_h.at[pl.ds(base, cpw)])
        pltpu.sync_copy(om, om_h.at[pl.ds(base, cpw)])

    return k(px, py, pz, c0, c1, c2, c3)


def _edge_feat_call(ep, src_dim, dst_dim, tabs0, tabs1, m1, s_idx, r_idx):
    epw = ep // NW
    C = 1280
    nchunks = epw // C
    n_out = 1 + (src_dim == 1) + (dst_dim == 1)

    src_tabs = tabs0 if src_dim == 0 else tabs1
    dst_tabs = tabs0 if dst_dim == 0 else tabs1
    n_src = N0 if src_dim == 0 else N1
    n_dst = N0 if dst_dim == 0 else N1
    same = src_dim == dst_dim

    scratch = [pltpu.VMEM((n_src,), F32)] * 3
    if not same:
        scratch += [pltpu.VMEM((n_dst,), F32)] * 3
    need_m1 = (src_dim == 1) or (dst_dim == 1)
    if need_m1:
        scratch += [pltpu.VMEM((N1,), F32)]
    scratch += [pltpu.VMEM((C,), jnp.int32)] * 2
    scratch += [pltpu.VMEM((C,), F32)] * n_out

    ins = list(src_tabs) + ([] if same else list(dst_tabs))
    if need_m1:
        ins.append(m1)
    ins += [s_idx, r_idx]

    @functools.partial(
        pl.kernel,
        out_type=[jax.ShapeDtypeStruct((ep,), F32)] * n_out,
        mesh=_vsmesh(),
        compiler_params=pltpu.CompilerParams(needs_layout_passes=False, use_tc_tiling_on_sc=False),
        scratch_types=scratch,
    )
    def k(*refs):
        pos = 0
        sx_h, sy_h, sz_h = refs[pos:pos + 3]; pos += 3
        if not same:
            dx_h, dy_h, dz_h = refs[pos:pos + 3]; pos += 3
        else:
            dx_h, dy_h, dz_h = sx_h, sy_h, sz_h
        if need_m1:
            m1_h = refs[pos]; pos += 1
        s_h, r_h = refs[pos:pos + 2]; pos += 2
        out_hs = refs[pos:pos + n_out]; pos += n_out
        sxv, syv, szv = refs[pos:pos + 3]; pos += 3
        if not same:
            dxv, dyv, dzv = refs[pos:pos + 3]; pos += 3
        else:
            dxv, dyv, dzv = sxv, syv, szv
        if need_m1:
            m1v = refs[pos]; pos += 1
        sv, rv = refs[pos:pos + 2]; pos += 2
        obufs = refs[pos:pos + n_out]

        wid = lax.axis_index("s") * NC + lax.axis_index("c")
        wbase = wid * epw
        pltpu.sync_copy(sx_h, sxv)
        pltpu.sync_copy(sy_h, syv)
        pltpu.sync_copy(sz_h, szv)
        if not same:
            pltpu.sync_copy(dx_h, dxv)
            pltpu.sync_copy(dy_h, dyv)
            pltpu.sync_copy(dz_h, dzv)
        if need_m1:
            pltpu.sync_copy(m1_h, m1v)

        for cc in range(nchunks):
            base = wbase + cc * C
            pltpu.sync_copy(s_h.at[pl.ds(base, C)], sv)
            pltpu.sync_copy(r_h.at[pl.ds(base, C)], rv)

            def body(t, carry):
                o = pl.multiple_of(t * 16, 16)
                si = sv[pl.ds(o, 16)]
                ri = rv[pl.ds(o, 16)]
                ax = plsc.load_gather(sxv, [si])
                ay = plsc.load_gather(syv, [si])
                az = plsc.load_gather(szv, [si])
                bx = plsc.load_gather(dxv, [ri])
                by = plsc.load_gather(dyv, [ri])
                bz = plsc.load_gather(dzv, [ri])
                ex = ax - bx
                ey = ay - by
                ez = az - bz
                ob = 0
                obufs[ob][pl.ds(o, 16)] = ex * ex + ey * ey + ez * ez
                ob += 1
                if src_dim == 1:
                    obufs[ob][pl.ds(o, 16)] = plsc.load_gather(m1v, [si])
                    ob += 1
                if dst_dim == 1:
                    obufs[ob][pl.ds(o, 16)] = plsc.load_gather(m1v, [ri])
                return carry

            lax.fori_loop(0, C // 16, body, 0)
            for b, oh in zip(obufs, out_hs):
                pltpu.sync_copy(b, oh.at[pl.ds(base, C)])

    return k(*ins)


def _stats_call(cols):
    n = len(cols)

    def body(*refs):
        in_refs = refs[:n]
        out_ref = refs[n]
        rows = lax.broadcasted_iota(jnp.int32, (8, 128), 0)
        colsq = lax.broadcasted_iota(jnp.int32, (8, 128), 1)
        acc = jnp.zeros((8, 128), F32)
        for c, ref in enumerate(in_refs):
            f = jnp.sqrt(ref[...] + 1e-12)
            s = jnp.sum(f)
            s2 = jnp.sum(f * f)
            acc = acc + jnp.where((rows == c) & (colsq == 0), s, 0.0)
            acc = acc + jnp.where((rows == c) & (colsq == 1), s2, 0.0)
        out_ref[...] = acc

    return pl.pallas_call(
        body,
        out_shape=jax.ShapeDtypeStruct((8, 128), F32),
    )(*cols)


def _tables0_call(x0, we, be, w00a, w00b, w01a):
    def body(x_ref, we_ref, be_ref, wa_ref, wb_ref, wc_ref,
             emb_ref, a00_ref, b00_ref, a01_ref):
        e = jnp.dot(x_ref[...], we_ref[...],
                    preferred_element_type=F32) + be_ref[...]
        emb_ref[...] = e
        a00_ref[...] = jnp.dot(e, wa_ref[...], preferred_element_type=F32)
        b00_ref[...] = jnp.dot(e, wb_ref[...], preferred_element_type=F32)
        a01_ref[...] = jnp.dot(e, wc_ref[...], preferred_element_type=F32)

    blk = 2000
    wspec = pl.BlockSpec((128, 128), lambda i: (0, 0))
    bspec = pl.BlockSpec((1, 128), lambda i: (0, 0))
    rspec = pl.BlockSpec((blk, 128), lambda i: (i, 0))
    return pl.pallas_call(
        body,
        grid=(N0 // blk,),
        in_specs=[rspec, wspec, bspec, wspec, wspec, wspec],
        out_specs=[rspec] * 4,
        out_shape=[jax.ShapeDtypeStruct((N0, H), F32)] * 4,
    )(x0, we, be, w00a, w00b, w01a)


def _tables1_call(x1, msq, sv, we, be, w01b, w11a, w11b, w01dr, w11ds, w11dr):
    def body(x_ref, m_ref, sv_ref, we_ref, be_ref, wb_ref, wa1_ref, wb1_ref,
             r01_ref, rs1_ref, rr1_ref,
             emb_ref, b01_ref, a11_ref, b11_ref):
        e = jnp.dot(x_ref[...], we_ref[...],
                    preferred_element_type=F32) + be_ref[...]
        emb_ref[...] = e
        d1 = jnp.sqrt(m_ref[...] + 1e-12)
        b01_ref[...] = (jnp.dot(e, wb_ref[...], preferred_element_type=F32)
                        + (d1 - sv_ref[0, 0]) * sv_ref[0, 1] * r01_ref[...])
        a11_ref[...] = (jnp.dot(e, wa1_ref[...], preferred_element_type=F32)
                        + (d1 - sv_ref[0, 2]) * sv_ref[0, 3] * rs1_ref[...])
        b11_ref[...] = (jnp.dot(e, wb1_ref[...], preferred_element_type=F32)
                        + (d1 - sv_ref[0, 4]) * sv_ref[0, 5] * rr1_ref[...])

    blk = 2000
    wspec = pl.BlockSpec((128, 128), lambda i: (0, 0))
    vspec = pl.BlockSpec((1, 128), lambda i: (0, 0))
    rspec = pl.BlockSpec((blk, 128), lambda i: (i, 0))
    mspec = pl.BlockSpec((blk, 1), lambda i: (i, 0))
    return pl.pallas_call(
        body,
        grid=(N1 // blk,),
        in_specs=[rspec, mspec, vspec, wspec, vspec, wspec, wspec, wspec,
                  vspec, vspec, vspec],
        out_specs=[rspec] * 4,
        out_shape=[jax.ShapeDtypeStruct((N1, H), F32)] * 4,
    )(x1, msq, sv, we, be, w01b, w11a, w11b, w01dr, w11ds, w11dr)


def _edge_gather_call(ep, a_tab, b_tab, s_idx, r_idx):
    epw = ep // NW
    K = 128
    nb = epw // K

    @functools.partial(
        pl.kernel,
        out_type=jax.ShapeDtypeStruct((ep, H), F32),
        mesh=_vsmesh(),
        compiler_params=pltpu.CompilerParams(needs_layout_passes=False, use_tc_tiling_on_sc=False),
        scratch_types=[
            pltpu.VMEM((epw,), jnp.int32),
            pltpu.VMEM((epw,), jnp.int32),
            [pltpu.VMEM((K, H), F32)] * 2,
            [pltpu.VMEM((K, H), F32)] * 2,
            [pltpu.SemaphoreType.DMA] * 2,
            [pltpu.SemaphoreType.DMA] * 2,
        ],
    )
    def k(a_h, b_h, s_h, r_h, out_h, sv, rv, bufa, bufb, gsem, osem):
        wid = lax.axis_index("s") * NC + lax.axis_index("c")
        wbase = wid * epw
        pltpu.sync_copy(s_h.at[pl.ds(wbase, epw)], sv)
        pltpu.sync_copy(r_h.at[pl.ds(wbase, epw)], rv)

        def issue(t, p):
            o = pl.ds(t * K, K)
            pltpu.async_copy(a_h.at[sv.at[o]], bufa[p], gsem[p])
            pltpu.async_copy(b_h.at[rv.at[o]], bufb[p], gsem[p])

        def drain_gather(t, p):
            o = pl.ds(t * K, K)
            pltpu.make_async_copy(a_h.at[sv.at[o]], bufa[p], gsem[p]).wait()
            pltpu.make_async_copy(b_h.at[rv.at[o]], bufb[p], gsem[p]).wait()

        def consume(t, p):
            base = wbase + t * K

            def addrow(i, c2):
                for c8 in range(8):
                    sl2 = pl.ds(c8 * 16, 16)
                    bufa[p][i, sl2] = bufa[p][i, sl2] + bufb[p][i, sl2]
                return c2

            lax.fori_loop(0, K, addrow, 0)
            pltpu.async_copy(bufa[p], out_h.at[pl.ds(base, K)], osem[p])

        issue(0, 0)

        def pair(i, carry):
            for sub in range(2):
                t = 2 * i + sub
                nxt = 1 - sub

                @pl.when(t + 1 < nb)
                def _():
                    @pl.when(t >= 1)
                    def _():
                        pltpu.make_async_copy(
                            bufa[nxt], out_h.at[pl.ds(0, K)], osem[nxt]).wait()
                    issue(t + 1, nxt)

                drain_gather(t, sub)
                consume(t, sub)
            return carry

        lax.fori_loop(0, nb // 2, pair, 0)
        pltpu.make_async_copy(bufa[0], out_h.at[pl.ds(0, K)], osem[0]).wait()
        pltpu.make_async_copy(bufa[1], out_h.at[pl.ds(0, K)], osem[1]).wait()

    return k(a_tab, b_tab, s_idx, r_idx)


def _edge_mlp_call(ep, e_real, h_sum, dsq_col, sv, wd, b1, w2, b2, wg):
    blk = 1024

    def body(h_ref, d_ref, sv_ref, wd_ref, b1_ref, w2_ref, b2_ref, wg_ref,
             lo_ref, hi_ref):
        i = pl.program_id(0)
        dn = (jnp.sqrt(d_ref[...] + 1e-12) - sv_ref[0, 0]) * sv_ref[0, 1]
        hp = h_ref[...] + dn * wd_ref[...] + b1_ref[...]
        hp = hp * jax.nn.sigmoid(hp)
        m = jnp.dot(hp, w2_ref[...], preferred_element_type=F32) + b2_ref[...]
        m = m * jax.nn.sigmoid(m)
        g = jax.nn.sigmoid(
            jnp.dot(m, wg_ref[...], preferred_element_type=F32) + sv_ref[0, 2])
        mg = m * g
        rows = i * blk + lax.broadcasted_iota(jnp.int32, (blk, 1), 0)
        mg = jnp.where(rows < e_real, mg, 0.0)
        lo_ref[...] = mg[:, :64]
        hi_ref[...] = mg[:, 64:]

    wspec = pl.BlockSpec((128, 128), lambda i: (0, 0))
    vspec = pl.BlockSpec((1, 128), lambda i: (0, 0))
    return pl.pallas_call(
        body,
        grid=(ep // blk,),
        in_specs=[pl.BlockSpec((blk, 128), lambda i: (i, 0)),
                  pl.BlockSpec((blk, 1), lambda i: (i, 0)),
                  vspec, vspec, vspec, wspec, vspec,
                  pl.BlockSpec((128, 1), lambda i: (0, 0))],
        out_specs=[pl.BlockSpec((blk, 64), lambda i: (i, 0))] * 2,
        out_shape=[jax.ShapeDtypeStruct((ep, 64), F32)] * 2,
    )(h_sum, dsq_col, sv, wd, b1, w2, b2, wg)


N0T = 632
N1T = 1256
N0SH = N0T * NS
N1SH = N1T * NS
KSC = 128


def _scatter_call(m00lo, m00hi, m01lo, m01hi, m11lo, m11hi,
                  r00, r01, r11, zer):
    @functools.partial(
        pl.kernel,
        out_type=[jax.ShapeDtypeStruct((N0SH, 64), F32)] * 2
        + [jax.ShapeDtypeStruct((N1SH, 64), F32)] * 2,
        mesh=_vsmesh(),
        compiler_params=pltpu.CompilerParams(needs_layout_passes=False, use_tc_tiling_on_sc=False),
        scratch_types=[
            pltpu.VMEM_SHARED((N1SH, 64), F32),
            pltpu.VMEM((E00P // NS // KSC, 128), jnp.int32),
            [pltpu.VMEM((KSC, 64), F32)] * 2,
            [pltpu.SemaphoreType.DMA] * 2,
        ],
    )
    def k(m00lo_h, m00hi_h, m01lo_h, m01hi_h, m11lo_h, m11hi_h,
          r00_h, r01_h, r11_h, zer_h,
          a0lo_h, a0hi_h, a1lo_h, a1hi_h,
          ash, idxv, mbuf, sem):
        c = lax.axis_index("c")
        s = lax.axis_index("s")

        def run(m_h, r_h, ep, ash):
            ept = ep // NS
            nb = ept // KSC
            pltpu.sync_copy(r_h.at[pl.ds(s * nb, nb)], idxv.at[pl.ds(0, nb)])

            def issue(t, p):
                base = s * ept + t * KSC
                pltpu.async_copy(m_h.at[pl.ds(base, KSC)], mbuf[p], sem[p])

            def drain(t, p):
                base = s * ept + t * KSC
                pltpu.make_async_copy(
                    m_h.at[pl.ds(base, KSC)], mbuf[p], sem[p]).wait()

            issue(0, 0)

            def pair(i, carry):
                for sub in range(2):
                    t = 2 * i + sub

                    @pl.when(t + 1 < nb)
                    def _():
                        issue(t + 1, 1 - sub)

                    drain(t, sub)
                    pltpu.sync_copy(mbuf[sub], ash.at[idxv.at[t]], add=True)
                return carry

            lax.fori_loop(0, nb // 2, pair, 0)

        pltpu.sync_copy(zer_h.at[pl.ds(0, N0T)], ash.at[pl.ds(s * N0T, N0T)])
        plsc.subcore_barrier()

        @pl.when(c == 0)
        def _():
            run(m00lo_h, r00_h, E00P, ash)

        @pl.when(c == 1)
        def _():
            run(m00hi_h, r00_h, E00P, ash)

        plsc.subcore_barrier()

        @pl.when(c == 0)
        def _():
            pltpu.sync_copy(ash.at[pl.ds(s * N0T, N0T)],
                            a0lo_h.at[pl.ds(s * N0T, N0T)])

        @pl.when(c == 1)
        def _():
            pltpu.sync_copy(ash.at[pl.ds(s * N0T, N0T)],
                            a0hi_h.at[pl.ds(s * N0T, N0T)])

        plsc.subcore_barrier()

        pltpu.sync_copy(zer_h, ash.at[pl.ds(s * N1T, N1T)])
        plsc.subcore_barrier()

        @pl.when(c == 0)
        def _():
            run(m01lo_h, r01_h, E01P, ash)
            run(m11lo_h, r11_h, E11P, ash)

        @pl.when(c == 1)
        def _():
            run(m01hi_h, r01_h, E01P, ash)
            run(m11hi_h, r11_h, E11P, ash)

        plsc.subcore_barrier()

        @pl.when(c == 0)
        def _():
            pltpu.sync_copy(ash.at[pl.ds(s * N1T, N1T)],
                            a1lo_h.at[pl.ds(s * N1T, N1T)])

        @pl.when(c == 1)
        def _():
            pltpu.sync_copy(ash.at[pl.ds(s * N1T, N1T)],
                            a1hi_h.at[pl.ds(s * N1T, N1T)])

    return k(m00lo, m00hi, m01lo, m01hi, m11lo, m11hi,
             r00, r01, r11, zer)


def _update_call(n, emb, agglo, agghi, wux, wualo, wuahi, bu, wp, bvec):
    blk = 2000

    def body(e_ref, lo_ref, hi_ref, wux_ref, wlo_ref, whi_ref, bu_ref,
             wp_ref, bv_ref, out_ref):
        e = e_ref[...]
        xn = (e + jnp.dot(e, wux_ref[...], preferred_element_type=F32)
              + jnp.dot(lo_ref[...], wlo_ref[...], preferred_element_type=F32)
              + jnp.dot(hi_ref[...], whi_ref[...], preferred_element_type=F32)
              + bu_ref[...])
        out_ref[...] = (jnp.dot(xn, wp_ref[...], preferred_element_type=F32)
                        + bv_ref[0, 0])

    wspec = pl.BlockSpec((128, 128), lambda i: (0, 0))
    hspec = pl.BlockSpec((64, 128), lambda i: (0, 0))
    vspec = pl.BlockSpec((1, 128), lambda i: (0, 0))
    return pl.pallas_call(
        body,
        grid=(n // blk,),
        in_specs=[pl.BlockSpec((blk, 128), lambda i: (i, 0)),
                  pl.BlockSpec((blk, 64), lambda i: (i, 0)),
                  pl.BlockSpec((blk, 64), lambda i: (i, 0)),
                  wspec, hspec, hspec, vspec,
                  pl.BlockSpec((128, 1), lambda i: (0, 0)), vspec],
        out_specs=pl.BlockSpec((blk, 1), lambda i: (i, 0)),
        out_shape=jax.ShapeDtypeStruct((n, 1), F32),
    )(emb, agglo, agghi, wux, wualo, wuahi, bu, wp, bvec)


def _pad1(x, n, val):
    return jnp.concatenate(
        [x, jnp.full((n - x.shape[0],), val, dtype=x.dtype)])


def _stat(sums, row, count):
    mu = sums[row, 0] / count
    var = sums[row, 1] / count - mu * mu
    isig = lax.rsqrt(var + 1e-5)
    return mu, isig


def kernel(pos, x_0, x_1, cell_1, adj_0_0, adj_0_1, adj_1_1, params):
    posx, posy, posz = pos[:, 0], pos[:, 1], pos[:, 2]
    cpad = jnp.pad(cell_1, ((0, N1P - N1), (0, 0)))
    c0, c1_, c2, c3 = (cpad[:, k] for k in range(4))

    s00 = _pad1(adj_0_0[0], E00P, 0)
    r00 = _pad1(adj_0_0[1], E00P, 0)
    s01 = _pad1(adj_0_1[0], E01P, 0)
    r01 = _pad1(adj_0_1[1], E01P, 0)
    s11 = _pad1(adj_1_1[0], E11P, 0)
    r11 = _pad1(adj_1_1[1], E11P, 0)

    ccx, ccy, ccz, cmsq = _geom_call(posx, posy, posz, c0, c1_, c2, c3)
    tabs0 = (posx, posy, posz)
    tabs1 = (ccx[:N1], ccy[:N1], ccz[:N1])
    msq1 = cmsq[:N1]

    (d00,) = _edge_feat_call(E00P, 0, 0, tabs0, tabs1, msq1, s00, r00)
    d01, dr01 = _edge_feat_call(E01P, 0, 1, tabs0, tabs1, msq1, s01, r01)
    d11, ds11, dr11 = _edge_feat_call(E11P, 1, 1, tabs0, tabs1, msq1, s11, r11)

    st00 = _stats_call([d00[:E00].reshape(-1, 128)])
    st01 = _stats_call([d01[:E01].reshape(-1, 128),
                        dr01[:E01].reshape(-1, 128)])
    st11 = _stats_call([d11[:E11].reshape(-1, 128),
                        ds11[:E11].reshape(-1, 128),
                        dr11[:E11].reshape(-1, 128)])

    mu00d, is00d = _stat(st00, 0, E00)
    mu01d, is01d = _stat(st01, 0, E01)
    mu01r, is01r = _stat(st01, 1, E01)
    mu11d, is11d = _stat(st11, 0, E11)
    mu11s, is11s = _stat(st11, 1, E11)
    mu11r, is11r = _stat(st11, 2, E11)

    p = params
    w1_00, w1_01, w1_11 = p["W1_0_0"], p["W1_0_1"], p["W1_1_1"]

    emb0, a00, b00, a01 = _tables0_call(
        x_0, p["W_emb_0"], p["b_emb_0"].reshape(1, H),
        w1_00[:H], w1_00[H:2 * H], w1_01[:H])
    sv1 = jnp.stack([mu01r, is01r, mu11s, is11s, mu11r, is11r])
    sv1 = jnp.pad(sv1, (0, 128 - 6)).reshape(1, 128)
    emb1, b01, a11, b11 = _tables1_call(
        x_1, msq1.reshape(N1, 1), sv1,
        p["W_emb_1"], p["b_emb_1"].reshape(1, H),
        w1_01[H:2 * H], w1_11[:H], w1_11[H:2 * H],
        w1_01[2 * H + 2].reshape(1, H),
        w1_11[2 * H + 1].reshape(1, H),
        w1_11[2 * H + 2].reshape(1, H))

    h00 = _edge_gather_call(E00P, a00, b00, s00, r00)
    h01 = _edge_gather_call(E01P, a01, b01, s01, r01)
    h11 = _edge_gather_call(E11P, a11, b11, s11, r11)

    def mlp(ep, e_real, h, dsq, mu, isig, a):
        sv = jnp.stack([mu, isig, p[f"bg_{a}"][0]])
        sv = jnp.pad(sv, (0, 125)).reshape(1, 128)
        return _edge_mlp_call(
            ep, e_real, h, dsq.reshape(ep, 1), sv,
            p[f"W1_{a}"][2 * H].reshape(1, H),
            p[f"b1_{a}"].reshape(1, H),
            p[f"W2_{a}"], p[f"b2_{a}"].reshape(1, H),
            p[f"Wg_{a}"])

    m00lo, m00hi = mlp(E00P, E00, h00, d00, mu00d, is00d, "0_0")
    m01lo, m01hi = mlp(E01P, E01, h01, d01, mu01d, is01d, "0_1")
    m11lo, m11hi = mlp(E11P, E11, h11, d11, mu11d, is11d, "1_1")

    zer = jnp.zeros((N1T, 64), F32)
    a0lo, a0hi, a1lo, a1hi = _scatter_call(
        m00lo, m00hi, m01lo, m01hi, m11lo, m11hi,
        r00.reshape(-1, 128), r01.reshape(-1, 128), r11.reshape(-1, 128),
        zer)
    a0lo, a0hi = a0lo[:N0], a0hi[:N0]
    a1lo, a1hi = a1lo[:N1], a1hi[:N1]

    wu0, wu1 = p["W_upd_0"], p["W_upd_1"]
    out0 = _update_call(
        N0, emb0, a0lo, a0hi, wu0[:H], wu0[H:H + 64], wu0[H + 64:],
        p["b_upd_0"].reshape(1, H), p["W_pre_0"],
        jnp.pad(p["b_pre_0"], (0, 127)).reshape(1, 128))
    out1 = _update_call(
        N1, emb1, a1lo, a1hi, wu1[:H], wu1[H:H + 64], wu1[H + 64:],
        p["b_upd_1"].reshape(1, H), p["W_pre_1"],
        jnp.pad(p["b_pre_1"], (0, 127)).reshape(1, 128))
    return out0, out1
